# Initial kernel scaffold; baseline (speedup 1.0000x reference)
import jax
import jax.numpy as jnp
from jax.experimental import pallas as pl


def kernel(speaker_labels, table):
    raise NotImplementedError("write your pallas kernel here")


# speedup vs baseline: 1.2933x; 1.2933x over previous
import functools

import jax
import jax.numpy as jnp
from jax import lax
from jax.experimental import pallas as pl
from jax.experimental.pallas import tpu as pltpu
from jax.experimental.pallas import tpu_sc as plsc

BATCH = 16384
HIST = 200
DIM = 64
N = BATCH * HIST
N2 = N // 2

NUM_CORES = 2
NUM_SUBCORES = 16
NUM_WORKERS = NUM_CORES * NUM_SUBCORES
PER_WORKER = N2 // NUM_WORKERS

CP = 256
SUB = 128
N_SUB = CP // SUB
GROUPS = CP // 16
CHUNKS_PER_WORKER = PER_WORKER // CP

_mesh = plsc.VectorSubcoreMesh(core_axis_name="c", subcore_axis_name="s")


@functools.partial(
    pl.kernel,
    mesh=_mesh,
    compiler_params=pltpu.CompilerParams(needs_layout_passes=False),
    out_type=jax.ShapeDtypeStruct((N2, 2 * DIM), jnp.float32),
    scratch_types=[
        pltpu.VMEM((2 * CP,), jnp.int32),
        pltpu.VMEM((CP,), jnp.int32),
        pltpu.VMEM((CP, 2 * DIM), jnp.float32),
        pltpu.SemaphoreType.DMA,
    ],
)
def _lookup(labels_hbm, pairs_hbm, out_hbm, raw_v, pid_v, rows_v, sem):
    wid = lax.axis_index("s") * NUM_CORES + lax.axis_index("c")
    base = wid * PER_WORKER
    lane = lax.iota(jnp.int32, 16)

    def chunk_body(i, carry):
        off = base + i * CP
        pltpu.sync_copy(labels_hbm.at[pl.ds(off * 2, 2 * CP)], raw_v)
        for g in range(GROUPS):
            ev = plsc.load_gather(raw_v, [lane * 2 + (32 * g)])
            od = plsc.load_gather(raw_v, [lane * 2 + (32 * g + 1)])
            pid_v[pl.ds(16 * g, 16)] = ev * 3 + od
        copies = [
            pltpu.async_copy(
                pairs_hbm.at[pid_v.at[pl.ds(j * SUB, SUB)]],
                rows_v.at[pl.ds(j * SUB, SUB)],
                sem,
            )
            for j in range(N_SUB)
        ]
        for c in copies:
            c.wait()
        pltpu.sync_copy(rows_v, out_hbm.at[pl.ds(off, CP)])
        return carry

    lax.fori_loop(0, CHUNKS_PER_WORKER, chunk_body, 0)


def kernel(speaker_labels, table):
    tbl = table.at[0].set(0.0)
    pair_tbl = jnp.concatenate(
        [
            jnp.repeat(tbl, 3, axis=0),
            jnp.tile(tbl, (3, 1)),
        ],
        axis=1,
    )
    flat = speaker_labels.reshape(N)
    out = _lookup(flat, pair_tbl)
    return out.reshape(BATCH, HIST, DIM)


# speedup vs baseline: 5.7694x; 4.4609x over previous
import functools

import jax
import jax.numpy as jnp
from jax import lax
from jax.experimental import pallas as pl
from jax.experimental.pallas import tpu as pltpu
from jax.experimental.pallas import tpu_sc as plsc

BATCH = 16384
HIST = 200
DIM = 64
N = BATCH * HIST
N2 = N // 2

NUM_CORES = 2
NUM_SUBCORES = 16
NUM_WORKERS = NUM_CORES * NUM_SUBCORES
PER_WORKER = N2 // NUM_WORKERS

CP = 400
GROUPS = CP // 16
CHUNKS = PER_WORKER // CP
SUBS = [(s, min(128, CP - s)) for s in range(0, CP, 128)]

_mesh = plsc.VectorSubcoreMesh(core_axis_name="c", subcore_axis_name="s")


@functools.partial(
    pl.kernel,
    mesh=_mesh,
    compiler_params=pltpu.CompilerParams(needs_layout_passes=False),
    out_type=jax.ShapeDtypeStruct((N2, 2 * DIM), jnp.float32),
    scratch_types=[
        pltpu.VMEM((2 * CP,), jnp.int32),
        pltpu.VMEM((2 * CP,), jnp.int32),
        pltpu.VMEM((CP,), jnp.int32),
        pltpu.VMEM((CP,), jnp.int32),
        pltpu.VMEM((CP, 2 * DIM), jnp.float32),
        pltpu.VMEM((CP, 2 * DIM), jnp.float32),
        pltpu.VMEM_SHARED((9, 2 * DIM), jnp.float32),
        pltpu.SemaphoreType.DMA,
        pltpu.SemaphoreType.DMA,
        pltpu.SemaphoreType.DMA,
        pltpu.SemaphoreType.DMA,
        pltpu.SemaphoreType.DMA,
    ],
)
def _lookup(
    labels_hbm,
    pairs_hbm,
    out_hbm,
    raw0,
    raw1,
    pid0,
    pid1,
    rows0,
    rows1,
    ptbl_sh,
    si0,
    si1,
    sg,
    so0,
    so1,
):
    raw = (raw0, raw1)
    pid = (pid0, pid1)
    rows = (rows0, rows1)
    si = (si0, si1)
    so = (so0, so1)

    sid = lax.axis_index("s")
    wid = sid * NUM_CORES + lax.axis_index("c")
    base = wid * PER_WORKER
    lane = lax.iota(jnp.int32, 16)

    @pl.when(sid == 0)
    def _():
        pltpu.sync_copy(pairs_hbm, ptbl_sh)

    plsc.subcore_barrier()

    pltpu.async_copy(labels_hbm.at[pl.ds(base * 2, 2 * CP)], raw[0], si[0])

    def half_step(i, p, q):
        off = base + i * CP
        pltpu.make_async_copy(
            labels_hbm.at[pl.ds(off * 2, 2 * CP)], raw[p], si[p]
        ).wait()

        @pl.when(i < CHUNKS - 1)
        def _():
            pltpu.async_copy(
                labels_hbm.at[pl.ds((off + CP) * 2, 2 * CP)], raw[q], si[q]
            )

        for g in range(GROUPS):
            ev = plsc.load_gather(raw[p], [lane * 2 + (32 * g)])
            od = plsc.load_gather(raw[p], [lane * 2 + (32 * g + 1)])
            pid[p][pl.ds(16 * g, 16)] = ev * 3 + od

        @pl.when(i >= 2)
        def _():
            pltpu.make_async_copy(
                rows[p], out_hbm.at[pl.ds(off, CP)], so[p]
            ).wait()

        copies = [
            pltpu.async_copy(
                ptbl_sh.at[pid[p].at[pl.ds(s, n)]],
                rows[p].at[pl.ds(s, n)],
                sg,
            )
            for s, n in SUBS
        ]
        for c in copies:
            c.wait()
        pltpu.async_copy(rows[p], out_hbm.at[pl.ds(off, CP)], so[p])

    def chunk_pair(i2, carry):
        half_step(i2 * 2, 0, 1)
        half_step(i2 * 2 + 1, 1, 0)
        return carry

    lax.fori_loop(0, CHUNKS // 2, chunk_pair, 0)

    for p in range(2):
        pltpu.make_async_copy(
            rows[p], out_hbm.at[pl.ds(base, CP)], so[p]
        ).wait()


def kernel(speaker_labels, table):
    tbl = table.at[0].set(0.0)
    pair_tbl = jnp.concatenate(
        [
            jnp.repeat(tbl, 3, axis=0),
            jnp.tile(tbl, (3, 1)),
        ],
        axis=1,
    )
    flat = speaker_labels.reshape(N)
    out = _lookup(flat, pair_tbl)
    return out.reshape(BATCH, HIST, DIM)
